# baseline (speedup 1.0000x reference)
import functools

import jax
import jax.numpy as jnp
from jax import lax
from jax.experimental import pallas as pl
from jax.experimental.pallas import tpu as pltpu
from jax.experimental.pallas import tpu_sc as plsc

L = 16
NW = 32
ROW = 128
CHUNK_ROWS = 6
NPHASE = 84


def _make_kernel(n_obs, n_points, n_cams):
    assert n_obs % ROW == 0
    n_rows = n_obs // ROW
    rows_base = n_rows // NW
    rows_extra = n_rows % NW
    chunk_obs = CHUNK_ROWS * ROW
    groups_per_chunk = chunk_obs // L
    assert (rows_base + 1 + CHUNK_ROWS - 1) // CHUNK_ROWS <= NPHASE
    assert NPHASE % 2 == 0

    mesh = plsc.VectorSubcoreMesh(core_axis_name="c", subcore_axis_name="s")

    plane_t = pltpu.VMEM((chunk_obs,), jnp.float32)
    buf_t = (
        [pltpu.VMEM((chunk_obs,), jnp.int32)] * 2
        + [plane_t] * 3
        + [plane_t] * 13
        + [pltpu.SemaphoreType.DMA] * 3
    )

    @functools.partial(
        pl.kernel,
        mesh=mesh,
        compiler_params=pltpu.CompilerParams(needs_layout_passes=False),
        out_type=tuple(jax.ShapeDtypeStruct((n_obs,), jnp.float32)
                       for _ in range(13)),
        scratch_types=[pltpu.VMEM((n_cams * 10,), jnp.float32)] + buf_t * 2,
    )
    def residual_kernel(cidx_hbm, pidx_hbm, ptx_hbm, pty_hbm, ptz_hbm,
                        cam_hbm, *rest):
        outs = rest[:13]
        cam_v = rest[13]
        bufs = rest[14:]
        A, B = bufs[:21], bufs[21:]
        w = lax.axis_index("s") * 2 + lax.axis_index("c")
        my_rows = rows_base + jnp.where(w < rows_extra, 1, 0)
        row_base = rows_base * w + jnp.minimum(w, rows_extra)

        pltpu.sync_copy(cam_hbm, cam_v)

        def base_ob(c):
            rb = row_base + jnp.minimum(c * CHUNK_ROWS, my_rows - CHUNK_ROWS)
            return rb * ROW

        def lin_issue(c, b):
            ob = base_ob(c)
            pltpu.async_copy(pidx_hbm.at[pl.ds(ob, chunk_obs)], b[0], b[18])
            pltpu.async_copy(cidx_hbm.at[pl.ds(ob, chunk_obs)], b[1], b[18])

        def lin_wait(b):
            pltpu.make_async_copy(
                pidx_hbm.at[pl.ds(0, chunk_obs)], b[0], b[18]).wait()
            pltpu.make_async_copy(
                cidx_hbm.at[pl.ds(0, chunk_obs)], b[1], b[18]).wait()

        def gather_fire(b):
            handles = []
            for j in range(CHUNK_ROWS):
                sl = pl.ds(j * ROW, ROW)
                idx = b[0].at[sl]
                for t, hbm in ((2, ptx_hbm), (3, pty_hbm), (4, ptz_hbm)):
                    handles.append(
                        pltpu.async_copy(hbm.at[idx], b[t].at[sl], b[19]))
            return handles

        def out_issue(c, b):
            ob = base_ob(c)
            for t in range(13):
                pltpu.async_copy(
                    b[5 + t], outs[t].at[pl.ds(ob, chunk_obs)], b[20])

        def out_wait(b):
            for t in range(13):
                pltpu.make_async_copy(
                    b[5 + t], outs[t].at[pl.ds(0, chunk_obs)], b[20]).wait()

        def stage_cam(b):
            def do_group(g, carry):
                sl = pl.ds(g * L, L)
                ci10 = b[1][sl] * 10
                for t in range(3):
                    b[5 + t][sl] = b[2 + t][sl]
                for t in range(10):
                    b[8 + t][sl] = plsc.load_gather(cam_v, [ci10 + t])
                return carry

            lax.fori_loop(0, groups_per_chunk, do_group, 0)

        def phase(c, cur, nxt, wait_out, comp):
            lin_wait(cur)
            handles = gather_fire(cur)
            if wait_out:
                out_wait(nxt)
            if comp:
                stage_cam(nxt)
                out_issue(c - 1, nxt)
            lin_issue(c + 1, nxt)
            for h in handles:
                h.wait()

        lin_issue(0, A)
        phase(jnp.int32(0), A, B, False, False)
        phase(jnp.int32(1), B, A, False, True)
        phase(jnp.int32(2), A, B, False, True)
        phase(jnp.int32(3), B, A, True, True)

        def pair(i, carry):
            c = 2 * i
            phase(c, A, B, True, True)
            phase(c + 1, B, A, True, True)
            return carry

        lax.fori_loop(2, NPHASE // 2, pair, 0)

        lin_wait(A)
        out_wait(B)
        stage_cam(B)
        out_issue(jnp.int32(NPHASE - 1), B)
        out_wait(A)
        out_wait(B)

    return residual_kernel


def kernel(observes, cidx, pidx, points, camera_params):
    n_obs = observes.shape[0]
    n_points, _ = points.shape
    n_cams, _ = camera_params.shape
    fn = _make_kernel(n_obs, n_points, n_cams)
    (px, py, pz, t0, t1, t2, qx, qy, qz, qw, fo, k1, k2) = fn(
        cidx.astype(jnp.int32), pidx.astype(jnp.int32),
        points[:, 0], points[:, 1], points[:, 2],
        camera_params.reshape(-1))

    two = jnp.float32(2.0)
    uvx = qy * pz - qz * py
    uvy = qz * px - qx * pz
    uvz = qx * py - qy * px
    uuvx = qy * uvz - qz * uvy
    uuvy = qz * uvx - qx * uvz
    uuvz = qx * uvy - qy * uvx
    cpx = (px + two * (qw * uvx + uuvx)) + t0
    cpy = (py + two * (qw * uvy + uuvy)) + t1
    cpz = (pz + two * (qw * uvz + uuvz)) + t2
    nx = (-cpx) / cpz
    ny = (-cpy) / cpz
    r2 = nx * nx + ny * ny
    dist = jnp.float32(1.0) + k1 * r2 + k2 * (r2 * r2)
    rx = fo * dist * nx - observes[:, 0]
    ry = fo * dist * ny - observes[:, 1]
    return jnp.stack([rx, ry], axis=-1)
